# Initial kernel scaffold; baseline (speedup 1.0000x reference)
import functools

import jax
import jax.numpy as jnp
from jax import lax
from jax.experimental import pallas as pl
from jax.experimental.pallas import tpu as pltpu
from jax.experimental.pallas import tpu_sc as plsc

_NUM_ROWS = 1000000
_DIM = 32
_BATCH = 16384
_HIST = 50
_TOTAL = _BATCH * _HIST
_NW = 32
_PER_W = _TOTAL // _NW
_IB = 128
_NG = _PER_W // _IB
_K = 8
_NCH = _NG // _K

_mesh = plsc.VectorSubcoreMesh(core_axis_name="c", subcore_axis_name="s")


@functools.partial(
    pl.kernel,
    mesh=_mesh,
    out_type=jax.ShapeDtypeStruct((_TOTAL, _DIM), jnp.float32),
    scratch_types=[
        pltpu.VMEM((_K, _IB), jnp.int32),
        pltpu.VMEM((_K * _IB, _DIM), jnp.float32),
        pltpu.SemaphoreType.DMA,
    ],
)
def _gather_kernel(weight_hbm, idx_hbm, out_hbm, idx_v, rows_v, sem):
    wid = lax.axis_index("s") * 2 + lax.axis_index("c")
    out_base = wid * _PER_W

    def chunk_body(ch, carry):
        g0 = ch * _K
        pltpu.sync_copy(idx_hbm.at[wid, pl.ds(g0, _K)], idx_v)
        copies = []
        for j in range(_K):
            copies.append(
                pltpu.async_copy(
                    weight_hbm.at[idx_v.at[j]],
                    rows_v.at[pl.ds(j * _IB, _IB)],
                    sem,
                )
            )
        for cp in copies:
            cp.wait()
        pltpu.sync_copy(
            rows_v, out_hbm.at[pl.ds(out_base + g0 * _IB, _K * _IB)]
        )
        return carry

    lax.fori_loop(0, _NCH, chunk_body, 0)


def kernel(x, weight):
    idx = x.astype(jnp.int32).reshape(_NW, _NG, _IB)
    out = _gather_kernel(weight, idx)
    return out.reshape(_BATCH, _HIST, _DIM)


# speedup vs baseline: 1.0943x; 1.0943x over previous
import functools

import jax
import jax.numpy as jnp
from jax import lax
from jax.experimental import pallas as pl
from jax.experimental.pallas import tpu as pltpu
from jax.experimental.pallas import tpu_sc as plsc

_NUM_ROWS = 1000000
_DIM = 32
_BATCH = 16384
_HIST = 50
_TOTAL = _BATCH * _HIST
_NW = 32
_PER_W = _TOTAL // _NW
_IB = 128
_NG = _PER_W // _IB
_K = 8
_NCH = _NG // _K

_mesh = plsc.VectorSubcoreMesh(core_axis_name="c", subcore_axis_name="s")


@functools.partial(
    pl.kernel,
    mesh=_mesh,
    out_type=jax.ShapeDtypeStruct((_TOTAL, _DIM), jnp.float32),
    scratch_types=[
        pltpu.VMEM((_K, _IB), jnp.int32),
        pltpu.VMEM((_K * _IB, _DIM), jnp.float32),
        pltpu.SemaphoreType.DMA,
    ],
    compiler_params=pltpu.CompilerParams(use_tc_tiling_on_sc=False),
)
def _gather_kernel(weight_hbm, idx_hbm, out_hbm, idx_v, rows_v, sem):
    wid = lax.axis_index("s") * 2 + lax.axis_index("c")
    out_base = wid * _PER_W

    def chunk_body(ch, carry):
        g0 = ch * _K
        pltpu.sync_copy(idx_hbm.at[wid, pl.ds(g0, _K)], idx_v)
        copies = []
        for j in range(_K):
            copies.append(
                pltpu.async_copy(
                    weight_hbm.at[idx_v.at[j]],
                    rows_v.at[pl.ds(j * _IB, _IB)],
                    sem,
                )
            )
        for cp in copies:
            cp.wait()
        pltpu.sync_copy(
            rows_v, out_hbm.at[pl.ds(out_base + g0 * _IB, _K * _IB)]
        )
        return carry

    lax.fori_loop(0, _NCH, chunk_body, 0)


def kernel(x, weight):
    idx = x.astype(jnp.int32).reshape(_NW, _NG, _IB)
    out = _gather_kernel(weight, idx)
    return out.reshape(_BATCH, _HIST, _DIM)


# speedup vs baseline: 1.1137x; 1.0177x over previous
import functools

import jax
import jax.numpy as jnp
from jax import lax
from jax.experimental import pallas as pl
from jax.experimental.pallas import tpu as pltpu
from jax.experimental.pallas import tpu_sc as plsc

_NUM_ROWS = 1000000
_DIM = 32
_BATCH = 16384
_HIST = 50
_TOTAL = _BATCH * _HIST
_NW = 32
_PER_W = _TOTAL // _NW
_IB = 128
_NG = _PER_W // _IB
_K = 10
_CROWS = _K * _IB
_NCH = _NG // _K

_mesh = plsc.VectorSubcoreMesh(core_axis_name="c", subcore_axis_name="s")


@functools.partial(
    pl.kernel,
    mesh=_mesh,
    out_type=jax.ShapeDtypeStruct((_TOTAL, _DIM), jnp.float32),
    scratch_types=[
        pltpu.VMEM((2, _K, _IB), jnp.int32),
        pltpu.VMEM((_CROWS, _DIM), jnp.float32),
        pltpu.VMEM((_CROWS, _DIM), jnp.float32),
        pltpu.SemaphoreType.DMA,
        pltpu.SemaphoreType.DMA,
        pltpu.SemaphoreType.DMA,
        pltpu.SemaphoreType.DMA,
    ],
    compiler_params=pltpu.CompilerParams(use_tc_tiling_on_sc=False),
)
def _gather_kernel(
    weight_hbm, idx_hbm, out_hbm, idx_v, rows0, rows1, gsem0, gsem1, wsem0, wsem1
):
    wid = lax.axis_index("s") * 2 + lax.axis_index("c")
    out_base = wid * _PER_W
    rows = (rows0, rows1)
    gsem = (gsem0, gsem1)
    wsem = (wsem0, wsem1)

    def load_idx(ch, b):
        pltpu.sync_copy(idx_hbm.at[wid, pl.ds(ch * _K, _K)], idx_v.at[b])

    def fire(b):
        for j in range(_K):
            pltpu.async_copy(
                weight_hbm.at[idx_v.at[b, j]],
                rows[b].at[pl.ds(j * _IB, _IB)],
                gsem[b],
            )

    def drain(b):
        for j in range(_K):
            pltpu.make_async_copy(
                weight_hbm.at[pl.ds(0, _IB)],
                rows[b].at[pl.ds(j * _IB, _IB)],
                gsem[b],
            ).wait()

    def start_wb(ch, b):
        pltpu.async_copy(
            rows[b], out_hbm.at[pl.ds(out_base + ch * _CROWS, _CROWS)], wsem[b]
        )

    def wait_wb(b):
        pltpu.make_async_copy(
            rows[b], out_hbm.at[pl.ds(out_base, _CROWS)], wsem[b]
        ).wait()

    load_idx(0, 0)
    fire(0)
    load_idx(1, 1)
    fire(1)

    def pair_body(p, carry):
        for b in range(2):
            ch = 2 * p + 2 + b
            drain(b)
            start_wb(ch - 2, b)
            load_idx(ch, b)
            wait_wb(b)
            fire(b)
        return carry

    lax.fori_loop(0, (_NCH - 2) // 2, pair_body, 0)

    drain(0)
    start_wb(_NCH - 2, 0)
    drain(1)
    start_wb(_NCH - 1, 1)
    wait_wb(0)
    wait_wb(1)


def kernel(x, weight):
    idx = x.astype(jnp.int32).reshape(_NW, _NG, _IB)
    out = _gather_kernel(weight, idx)
    return out.reshape(_BATCH, _HIST, _DIM)


# speedup vs baseline: 1.7939x; 1.6108x over previous
import functools

import jax
import jax.numpy as jnp
from jax import lax
from jax.experimental import pallas as pl
from jax.experimental.pallas import tpu as pltpu
from jax.experimental.pallas import tpu_sc as plsc

_NUM_ROWS = 1000000
_DIM = 32
_BATCH = 16384
_HIST = 50
_NW = 32
_XROWS_W = _BATCH // _NW
_R = 8
_NCH = _XROWS_W // _R

_mesh = plsc.VectorSubcoreMesh(core_axis_name="c", subcore_axis_name="s")


@functools.partial(
    pl.kernel,
    mesh=_mesh,
    out_type=jax.ShapeDtypeStruct((_BATCH, _HIST, _DIM), jnp.float32),
    scratch_types=[
        pltpu.VMEM((2, _R, _HIST), jnp.int32),
        pltpu.VMEM((_R, _HIST, _DIM), jnp.float32),
        pltpu.VMEM((_R, _HIST, _DIM), jnp.float32),
        pltpu.SemaphoreType.DMA,
        pltpu.SemaphoreType.DMA,
        pltpu.SemaphoreType.DMA,
        pltpu.SemaphoreType.DMA,
    ],
    compiler_params=pltpu.CompilerParams(use_tc_tiling_on_sc=False),
)
def _gather_kernel(
    weight_hbm, x_hbm, out_hbm, idx_v, rows0, rows1, gsem0, gsem1, wsem0, wsem1
):
    wid = lax.axis_index("s") * 2 + lax.axis_index("c")
    row_base = wid * _XROWS_W
    rows = (rows0, rows1)
    gsem = (gsem0, gsem1)
    wsem = (wsem0, wsem1)

    def load_idx(ch, b):
        pltpu.sync_copy(x_hbm.at[pl.ds(row_base + ch * _R, _R)], idx_v.at[b])

    def fire(b):
        for j in range(_R):
            pltpu.async_copy(
                weight_hbm.at[idx_v.at[b, j]], rows[b].at[j], gsem[b]
            )

    def drain(b):
        for j in range(_R):
            pltpu.make_async_copy(
                weight_hbm.at[pl.ds(0, _HIST)], rows[b].at[j], gsem[b]
            ).wait()

    def start_wb(ch, b):
        pltpu.async_copy(
            rows[b], out_hbm.at[pl.ds(row_base + ch * _R, _R)], wsem[b]
        )

    def wait_wb(b):
        pltpu.make_async_copy(
            rows[b], out_hbm.at[pl.ds(row_base, _R)], wsem[b]
        ).wait()

    load_idx(0, 0)
    fire(0)
    load_idx(1, 1)
    fire(1)

    def pair_body(p, carry):
        for b in range(2):
            ch = 2 * p + 2 + b
            drain(b)
            start_wb(ch - 2, b)
            load_idx(ch, b)
            wait_wb(b)
            fire(b)
        return carry

    lax.fori_loop(0, (_NCH - 2) // 2, pair_body, 0)

    drain(0)
    start_wb(_NCH - 2, 0)
    drain(1)
    start_wb(_NCH - 1, 1)
    wait_wb(0)
    wait_wb(1)


def kernel(x, weight):
    return _gather_kernel(weight, x.astype(jnp.int32))
